# baseline (speedup 1.0000x reference)
import math
import numpy as np
import jax
import jax.numpy as jnp
from jax.experimental import pallas as pl
from jax.experimental.pallas import tpu as pltpu

_BINS = 10
_CH = 32
_SG = 8


def _logit_edges():
    out = []
    for i in range(1, _BINS):
        e = float(np.float32(np.float32(i) / np.float32(_BINS)))
        out.append(np.float32(math.log(e / (1.0 - e))))
    return out


_EDGES_T = _logit_edges()


def _csum(x):
    return (x[0:_SG] + x[_SG:2 * _SG]) + (x[2 * _SG:3 * _SG] + x[3 * _SG:4 * _SG])


def _ghm_body(pred_ref, tgt_ref, w_ref, out_ref, acc_ref, tgt_scr):
    i = pl.program_id(0)
    nblk = pl.num_programs(0)
    nedge = _BINS - 1

    @pl.when(i == 0)
    def _init():
        for k in range(2 * nedge + 2):
            acc_ref[k] = 0.0

    nrow, ncls = pred_ref.shape
    tgt_scr[...] = tgt_ref[...].reshape(nrow, 1)
    wvec = w_ref[...].reshape(1, ncls)
    cls = jax.lax.broadcasted_iota(jnp.int32, (1, ncls), 1)

    def chunk(c, carry):
        base = c * _CH
        p = pred_ref[pl.ds(base, _CH), :]
        tg = tgt_scr[pl.ds(base, _CH), :]
        is_t = tg == cls
        ps = jnp.where(is_t, -p, p)
        loss = jnp.maximum(ps, 0.0) + jnp.log1p(jnp.exp(-jnp.abs(ps)))
        w_row = jnp.sum(jnp.where(is_t, wvec, 0.0), axis=1, keepdims=True)
        wl = w_row * loss
        out = []
        for k, t in enumerate(_EDGES_T):
            m = ps >= t
            out.append(carry[k] + _csum(jnp.where(m, 1.0, 0.0)))
        for k, t in enumerate(_EDGES_T):
            m = ps >= t
            out.append(carry[nedge + k] + _csum(jnp.where(m, wl, 0.0)))
        out.append(carry[2 * nedge] + _csum(wl))
        out.append(carry[2 * nedge + 1] + _csum(w_row))
        return tuple(out)

    zeros = tuple(jnp.zeros((_SG, ncls), jnp.float32) for _ in range(2 * nedge + 1)
                  ) + (jnp.zeros((_SG, 1), jnp.float32),)
    res = jax.lax.fori_loop(0, nrow // _CH, chunk, zeros)
    for k in range(2 * nedge + 2):
        acc_ref[k] = acc_ref[k] + jnp.sum(res[k])

    @pl.when(i == nblk - 1)
    def _finalize():
        tot = jnp.float32(nrow) * jnp.float32(nblk) * jnp.float32(ncls)
        ccum = [tot] + [acc_ref[k] for k in range(nedge)] + [jnp.float32(0.0)]
        scum = ([acc_ref[2 * nedge]] + [acc_ref[nedge + k] for k in range(nedge)]
                + [jnp.float32(0.0)])
        n = jnp.float32(0.0)
        t = jnp.float32(0.0)
        for b in range(_BINS):
            cnt_b = ccum[b] - ccum[b + 1]
            s_b = jnp.where(cnt_b > 0.0, scum[b] - scum[b + 1], 0.0)
            n = n + (cnt_b > 0.0).astype(jnp.float32)
            t = t + s_b / jnp.maximum(cnt_b, 1.0)
        sumw = acc_ref[2 * nedge + 1] * jnp.float32(ncls)
        scaled = (tot / jnp.maximum(n, 1.0)) * t
        out_ref[0, 0] = jnp.where(n > 0.0, scaled, t) / sumw


def kernel(pred, target, W):
    nrows, ncls = pred.shape
    grid = 8
    rblk = nrows // grid

    out = pl.pallas_call(
        _ghm_body,
        grid=(grid,),
        in_specs=[
            pl.BlockSpec((rblk, ncls), lambda i: (i, 0)),
            pl.BlockSpec((rblk,), lambda i: (i,)),
            pl.BlockSpec((ncls,), lambda i: (0,)),
        ],
        out_specs=pl.BlockSpec(memory_space=pltpu.SMEM),
        out_shape=jax.ShapeDtypeStruct((1, 1), jnp.float32),
        scratch_shapes=[
            pltpu.SMEM((2 * _BINS,), jnp.float32),
            pltpu.VMEM((rblk, 1), jnp.int32),
        ],
        compiler_params=pltpu.CompilerParams(
            dimension_semantics=("arbitrary",)),
    )(pred, target, W)
    return out[0, 0]


# speedup vs baseline: 2.9251x; 2.9251x over previous
import math
import numpy as np
import jax
import jax.numpy as jnp
from jax.experimental import pallas as pl
from jax.experimental.pallas import tpu as pltpu

_BINS = 10
_CH = 32
_SG = 8


def _logit_edges():
    out = []
    for i in range(1, _BINS):
        e = float(np.float32(np.float32(i) / np.float32(_BINS)))
        out.append(np.float32(math.log(e / (1.0 - e))))
    return out


_EDGES_T = _logit_edges()


def _csum(x):
    return (x[0:_SG] + x[_SG:2 * _SG]) + (x[2 * _SG:3 * _SG] + x[3 * _SG:4 * _SG])


def _ghm_body(pred_ref, tgt_ref, w_ref, out_ref, acc_ref):
    i = pl.program_id(0)
    nblk = pl.num_programs(0)
    nedge = _BINS - 1

    @pl.when(i == 0)
    def _init():
        for k in range(2 * nedge + 2):
            acc_ref[k] = 0.0

    nrow, ncls = pred_ref.shape
    pred = pred_ref[...]
    tgt = tgt_ref[...].reshape(nrow, 1)
    wvec = w_ref[...].reshape(1, ncls)
    cls = jax.lax.broadcasted_iota(jnp.int32, (1, ncls), 1)

    is_t = tgt == cls
    ps = jnp.where(is_t, -pred, pred)
    loss = jnp.maximum(ps, 0.0) + jnp.log1p(jnp.exp(-jnp.abs(ps)))
    w_row = jnp.sum(jnp.where(is_t, wvec, 0.0), axis=1, keepdims=True)
    wl = w_row * loss

    padc = 128 - ncls
    psp = jnp.concatenate(
        [ps, jnp.full((nrow, padc), 1e9, jnp.float32)], axis=1)
    wlp = jnp.concatenate(
        [wl, jnp.zeros((nrow, padc), jnp.float32)], axis=1)

    for k, t in enumerate(_EDGES_T):
        m = psp >= t
        acc_ref[k] = acc_ref[k] + jnp.sum(jnp.sum(
            jnp.where(m, 1.0, 0.0), axis=0))
        acc_ref[nedge + k] = acc_ref[nedge + k] + jnp.sum(jnp.sum(
            jnp.where(m, wlp, 0.0), axis=0))
    acc_ref[2 * nedge] = acc_ref[2 * nedge] + jnp.sum(jnp.sum(wlp, axis=0))
    acc_ref[2 * nedge + 1] = acc_ref[2 * nedge + 1] + jnp.sum(w_row)

    @pl.when(i == nblk - 1)
    def _finalize():
        tot = jnp.float32(nrow) * jnp.float32(nblk) * jnp.float32(ncls)
        padcnt = jnp.float32(nrow) * jnp.float32(nblk) * jnp.float32(128 - ncls)
        ccum = ([tot] + [acc_ref[k] - padcnt for k in range(nedge)]
                + [jnp.float32(0.0)])
        scum = ([acc_ref[2 * nedge]] + [acc_ref[nedge + k] for k in range(nedge)]
                + [jnp.float32(0.0)])
        n = jnp.float32(0.0)
        t = jnp.float32(0.0)
        for b in range(_BINS):
            cnt_b = ccum[b] - ccum[b + 1]
            s_b = jnp.where(cnt_b > 0.0, scum[b] - scum[b + 1], 0.0)
            n = n + (cnt_b > 0.0).astype(jnp.float32)
            t = t + s_b / jnp.maximum(cnt_b, 1.0)
        sumw = acc_ref[2 * nedge + 1] * jnp.float32(ncls)
        scaled = (tot / jnp.maximum(n, 1.0)) * t
        out_ref[0, 0] = jnp.where(n > 0.0, scaled, t) / sumw


def kernel(pred, target, W):
    nrows, ncls = pred.shape
    grid = 8
    rblk = nrows // grid

    out = pl.pallas_call(
        _ghm_body,
        grid=(grid,),
        in_specs=[
            pl.BlockSpec((rblk, ncls), lambda i: (i, 0)),
            pl.BlockSpec((rblk,), lambda i: (i,)),
            pl.BlockSpec((ncls,), lambda i: (0,)),
        ],
        out_specs=pl.BlockSpec(memory_space=pltpu.SMEM),
        out_shape=jax.ShapeDtypeStruct((1, 1), jnp.float32),
        scratch_shapes=[pltpu.SMEM((2 * _BINS,), jnp.float32)],
        compiler_params=pltpu.CompilerParams(
            dimension_semantics=("arbitrary",)),
    )(pred, target, W)
    return out[0, 0]


# speedup vs baseline: 3.0054x; 1.0274x over previous
import math
import numpy as np
import jax
import jax.numpy as jnp
from jax.experimental import pallas as pl
from jax.experimental.pallas import tpu as pltpu

_BINS = 10
_CH = 32
_SG = 8


def _logit_edges():
    out = []
    for i in range(1, _BINS):
        e = float(np.float32(np.float32(i) / np.float32(_BINS)))
        out.append(np.float32(math.log(e / (1.0 - e))))
    return out


_EDGES_T = _logit_edges()


def _csum(x):
    return (x[0:_SG] + x[_SG:2 * _SG]) + (x[2 * _SG:3 * _SG] + x[3 * _SG:4 * _SG])


def _ghm_body(pred_ref, tgt_ref, w_ref, out_ref, acc_ref):
    i = pl.program_id(0)
    nblk = pl.num_programs(0)
    nedge = _BINS - 1

    @pl.when(i == 0)
    def _init():
        for k in range(2 * nedge + 2):
            acc_ref[k] = 0.0

    nrow, ncls = pred_ref.shape
    pred = pred_ref[...]
    tgt = tgt_ref[...].reshape(nrow, 1)
    wvec = w_ref[...].reshape(1, ncls)
    cls = jax.lax.broadcasted_iota(jnp.int32, (1, ncls), 1)

    is_t = tgt == cls
    ps = jnp.where(is_t, -pred, pred)
    loss = jnp.maximum(ps, 0.0) + jnp.log1p(jnp.exp(-jnp.abs(ps)))
    w_row = jnp.sum(jnp.where(is_t, wvec, 0.0), axis=1, keepdims=True)
    wl = w_row * loss

    padc = 128 - ncls
    psp = jnp.concatenate(
        [ps, jnp.full((nrow, padc), 1e9, jnp.float32)], axis=1)
    wlp = jnp.concatenate(
        [wl, jnp.zeros((nrow, padc), jnp.float32)], axis=1)

    for k, t in enumerate(_EDGES_T):
        m = psp >= t
        acc_ref[k] = acc_ref[k] + jnp.sum(jnp.sum(
            jnp.where(m, 1.0, 0.0), axis=0))
        acc_ref[nedge + k] = acc_ref[nedge + k] + jnp.sum(jnp.sum(
            jnp.where(m, wlp, 0.0), axis=0))
    acc_ref[2 * nedge] = acc_ref[2 * nedge] + jnp.sum(jnp.sum(wlp, axis=0))
    acc_ref[2 * nedge + 1] = acc_ref[2 * nedge + 1] + jnp.sum(w_row)

    @pl.when(i == nblk - 1)
    def _finalize():
        tot = jnp.float32(nrow) * jnp.float32(nblk) * jnp.float32(ncls)
        padcnt = jnp.float32(nrow) * jnp.float32(nblk) * jnp.float32(128 - ncls)
        ccum = ([tot] + [acc_ref[k] - padcnt for k in range(nedge)]
                + [jnp.float32(0.0)])
        scum = ([acc_ref[2 * nedge]] + [acc_ref[nedge + k] for k in range(nedge)]
                + [jnp.float32(0.0)])
        n = jnp.float32(0.0)
        t = jnp.float32(0.0)
        for b in range(_BINS):
            cnt_b = ccum[b] - ccum[b + 1]
            s_b = jnp.where(cnt_b > 0.0, scum[b] - scum[b + 1], 0.0)
            n = n + (cnt_b > 0.0).astype(jnp.float32)
            t = t + s_b / jnp.maximum(cnt_b, 1.0)
        sumw = acc_ref[2 * nedge + 1] * jnp.float32(ncls)
        scaled = (tot / jnp.maximum(n, 1.0)) * t
        out_ref[0, 0] = jnp.where(n > 0.0, scaled, t) / sumw


def kernel(pred, target, W):
    nrows, ncls = pred.shape
    grid = 4
    rblk = nrows // grid

    out = pl.pallas_call(
        _ghm_body,
        grid=(grid,),
        in_specs=[
            pl.BlockSpec((rblk, ncls), lambda i: (i, 0)),
            pl.BlockSpec((rblk,), lambda i: (i,)),
            pl.BlockSpec((ncls,), lambda i: (0,)),
        ],
        out_specs=pl.BlockSpec(memory_space=pltpu.SMEM),
        out_shape=jax.ShapeDtypeStruct((1, 1), jnp.float32),
        scratch_shapes=[pltpu.SMEM((2 * _BINS,), jnp.float32)],
        compiler_params=pltpu.CompilerParams(
            dimension_semantics=("arbitrary",)),
    )(pred, target, W)
    return out[0, 0]


# speedup vs baseline: 3.0320x; 1.0089x over previous
import math
import numpy as np
import jax
import jax.numpy as jnp
from jax.experimental import pallas as pl
from jax.experimental.pallas import tpu as pltpu

_BINS = 10
_CH = 32
_SG = 8


def _logit_edges():
    out = []
    for i in range(1, _BINS):
        e = float(np.float32(np.float32(i) / np.float32(_BINS)))
        out.append(np.float32(math.log(e / (1.0 - e))))
    return out


_EDGES_T = _logit_edges()


def _csum(x):
    return (x[0:_SG] + x[_SG:2 * _SG]) + (x[2 * _SG:3 * _SG] + x[3 * _SG:4 * _SG])


def _ghm_body(pred_ref, tgt_ref, w_ref, out_ref, acc_ref):
    i = pl.program_id(0)
    nblk = pl.num_programs(0)
    nedge = _BINS - 1

    @pl.when(i == 0)
    def _init():
        for k in range(2 * nedge + 2):
            acc_ref[k] = 0.0

    nrow, ncls = pred_ref.shape
    pred = pred_ref[...]
    tgt = tgt_ref[...].reshape(nrow, 1)
    wvec = w_ref[...].reshape(1, ncls)
    cls = jax.lax.broadcasted_iota(jnp.int32, (1, ncls), 1)

    is_t = tgt == cls
    ps = jnp.where(is_t, -pred, pred)
    loss = jnp.maximum(ps, 0.0) + jnp.log1p(jnp.exp(-jnp.abs(ps)))
    w_row = jnp.sum(jnp.where(is_t, wvec, 0.0), axis=1, keepdims=True)
    wl = w_row * loss

    padc = 128 - ncls
    psp = jnp.concatenate(
        [ps, jnp.full((nrow, padc), 1e9, jnp.float32)], axis=1)
    wlp = jnp.concatenate(
        [wl, jnp.zeros((nrow, padc), jnp.float32)], axis=1)

    for k, t in enumerate(_EDGES_T):
        m = psp >= t
        acc_ref[k] = acc_ref[k] + jnp.sum(jnp.sum(
            jnp.where(m, 1.0, 0.0), axis=0))
        acc_ref[nedge + k] = acc_ref[nedge + k] + jnp.sum(jnp.sum(
            jnp.where(m, wlp, 0.0), axis=0))
    acc_ref[2 * nedge] = acc_ref[2 * nedge] + jnp.sum(jnp.sum(wlp, axis=0))
    acc_ref[2 * nedge + 1] = acc_ref[2 * nedge + 1] + jnp.sum(w_row)

    @pl.when(i == nblk - 1)
    def _finalize():
        tot = jnp.float32(nrow) * jnp.float32(nblk) * jnp.float32(ncls)
        padcnt = jnp.float32(nrow) * jnp.float32(nblk) * jnp.float32(128 - ncls)
        ccum = ([tot] + [acc_ref[k] - padcnt for k in range(nedge)]
                + [jnp.float32(0.0)])
        scum = ([acc_ref[2 * nedge]] + [acc_ref[nedge + k] for k in range(nedge)]
                + [jnp.float32(0.0)])
        n = jnp.float32(0.0)
        t = jnp.float32(0.0)
        for b in range(_BINS):
            cnt_b = ccum[b] - ccum[b + 1]
            s_b = jnp.where(cnt_b > 0.0, scum[b] - scum[b + 1], 0.0)
            n = n + (cnt_b > 0.0).astype(jnp.float32)
            t = t + s_b / jnp.maximum(cnt_b, 1.0)
        sumw = acc_ref[2 * nedge + 1] * jnp.float32(ncls)
        scaled = (tot / jnp.maximum(n, 1.0)) * t
        out_ref[0, 0] = jnp.where(n > 0.0, scaled, t) / sumw


def kernel(pred, target, W):
    nrows, ncls = pred.shape
    grid = 2
    rblk = nrows // grid

    out = pl.pallas_call(
        _ghm_body,
        grid=(grid,),
        in_specs=[
            pl.BlockSpec((rblk, ncls), lambda i: (i, 0)),
            pl.BlockSpec((rblk,), lambda i: (i,)),
            pl.BlockSpec((ncls,), lambda i: (0,)),
        ],
        out_specs=pl.BlockSpec(memory_space=pltpu.SMEM),
        out_shape=jax.ShapeDtypeStruct((1, 1), jnp.float32),
        scratch_shapes=[pltpu.SMEM((2 * _BINS,), jnp.float32)],
        compiler_params=pltpu.CompilerParams(
            dimension_semantics=("arbitrary",)),
    )(pred, target, W)
    return out[0, 0]
